# initial kernel scaffold (unmeasured)
import jax
import jax.numpy as jnp
from jax import lax
from jax.experimental import pallas as pl
from jax.experimental.pallas import tpu as pltpu


def kernel(
    x,
):
    def body(*refs):
        pass

    out_shape = jax.ShapeDtypeStruct(..., jnp.float32)
    return pl.pallas_call(body, out_shape=out_shape)(...)


# baseline (device time: 26123 ns/iter reference)
import jax
import jax.numpy as jnp
from jax import lax
from jax.experimental import pallas as pl
from jax.experimental.pallas import tpu as pltpu

Z = 4


def kernel(x):
    m, n = x.shape
    blk = n // Z

    def body(x_ref, out_ref, xb_ref, send_sems, recv_sems):
        my_x = lax.axis_index("x")
        my_y = lax.axis_index("y")
        my_z = lax.axis_index("z")

        xb_ref[...] = x_ref[...].astype(jnp.bfloat16)

        barrier_sem = pltpu.get_barrier_semaphore()
        for d in range(1, Z):
            tz = (my_z + d) % Z
            pl.semaphore_signal(
                barrier_sem, inc=1,
                device_id=(my_x, my_y, tz),
                device_id_type=pl.DeviceIdType.MESH,
            )
        pl.semaphore_wait(barrier_sem, Z - 1)

        out_ref[pl.ds(my_z * m, m), :] = xb_ref[:, pl.ds(my_z * blk, blk)]

        rdmas = []
        for d in range(1, Z):
            tz = (my_z + d) % Z
            rdma = pltpu.make_async_remote_copy(
                src_ref=xb_ref.at[:, pl.ds(tz * blk, blk)],
                dst_ref=out_ref.at[pl.ds(my_z * m, m), :],
                send_sem=send_sems.at[d - 1],
                recv_sem=recv_sems.at[d - 1],
                device_id=(my_x, my_y, tz),
                device_id_type=pl.DeviceIdType.MESH,
            )
            rdma.start()
            rdmas.append(rdma)

        for rdma in rdmas:
            rdma.wait_send()
        for rdma in rdmas:
            rdma.wait_recv()

    out_shape = jax.ShapeDtypeStruct((Z * m, blk), jnp.bfloat16)
    return pl.pallas_call(
        body,
        out_shape=out_shape,
        in_specs=[pl.BlockSpec(memory_space=pltpu.VMEM)],
        out_specs=pl.BlockSpec(memory_space=pltpu.VMEM),
        scratch_shapes=[
            pltpu.VMEM((m, n), jnp.bfloat16),
            pltpu.SemaphoreType.DMA((Z - 1,)),
            pltpu.SemaphoreType.DMA((Z - 1,)),
        ],
        compiler_params=pltpu.CompilerParams(collective_id=0),
    )(x)


# device time: 25346 ns/iter; 1.0307x vs baseline; 1.0307x over previous
import jax
import jax.numpy as jnp
from jax import lax
from jax.experimental import pallas as pl
from jax.experimental.pallas import tpu as pltpu

Z = 4


def kernel(x):
    m, n = x.shape
    blk = n // Z
    qm = m // 4
    hm = m // 2

    def body(x_ref, out_ref, xb_ref,
             zs_sems, zr_sems, xs_sems, xr_sems, ys_sems, yr_sems):
        my_x = lax.axis_index("x")
        my_y = lax.axis_index("y")
        my_z = lax.axis_index("z")
        q = 2 * my_y + my_x

        xb_ref[...] = x_ref[...].astype(jnp.bfloat16)

        barrier_sem = pltpu.get_barrier_semaphore()
        for d in range(1, Z):
            pl.semaphore_signal(
                barrier_sem, inc=1,
                device_id=(my_x, my_y, (my_z + d) % Z),
                device_id_type=pl.DeviceIdType.MESH,
            )
        for nbr in ((1 - my_x, my_y, my_z), (my_x, 1 - my_y, my_z)):
            pl.semaphore_signal(
                barrier_sem, inc=1,
                device_id=nbr, device_id_type=pl.DeviceIdType.MESH,
            )
        pl.semaphore_wait(barrier_sem, Z + 1)

        z_rdmas = []
        for d in range(1, Z):
            tz = (my_z + d) % Z
            r = pltpu.make_async_remote_copy(
                src_ref=xb_ref.at[pl.ds(q * qm, qm), pl.ds(tz * blk, blk)],
                dst_ref=out_ref.at[pl.ds(my_z * m + q * qm, qm), :],
                send_sem=zs_sems.at[d - 1],
                recv_sem=zr_sems.at[d - 1],
                device_id=(my_x, my_y, tz),
                device_id_type=pl.DeviceIdType.MESH,
            )
            r.start()
            z_rdmas.append(r)

        out_ref[pl.ds(my_z * m, m), :] = xb_ref[:, pl.ds(my_z * blk, blk)]

        x_rdmas = []
        for d in range(1, Z):
            sz = (my_z - d) % Z
            z_rdmas[d - 1].wait_recv()
            r = pltpu.make_async_remote_copy(
                src_ref=out_ref.at[pl.ds(sz * m + q * qm, qm), :],
                dst_ref=out_ref.at[pl.ds(sz * m + q * qm, qm), :],
                send_sem=xs_sems.at[d - 1],
                recv_sem=xr_sems.at[d - 1],
                device_id=(1 - my_x, my_y, my_z),
                device_id_type=pl.DeviceIdType.MESH,
            )
            r.start()
            x_rdmas.append(r)

        y_rdmas = []
        for d in range(1, Z):
            sz = (my_z - d) % Z
            x_rdmas[d - 1].wait_recv()
            r = pltpu.make_async_remote_copy(
                src_ref=out_ref.at[pl.ds(sz * m + my_y * hm, hm), :],
                dst_ref=out_ref.at[pl.ds(sz * m + my_y * hm, hm), :],
                send_sem=ys_sems.at[d - 1],
                recv_sem=yr_sems.at[d - 1],
                device_id=(my_x, 1 - my_y, my_z),
                device_id_type=pl.DeviceIdType.MESH,
            )
            r.start()
            y_rdmas.append(r)

        for r in y_rdmas:
            r.wait_recv()
        for r in z_rdmas + x_rdmas + y_rdmas:
            r.wait_send()

    out_shape = jax.ShapeDtypeStruct((Z * m, blk), jnp.bfloat16)
    return pl.pallas_call(
        body,
        out_shape=out_shape,
        in_specs=[pl.BlockSpec(memory_space=pltpu.VMEM)],
        out_specs=pl.BlockSpec(memory_space=pltpu.VMEM),
        scratch_shapes=[
            pltpu.VMEM((m, n), jnp.bfloat16),
            pltpu.SemaphoreType.DMA((Z - 1,)),
            pltpu.SemaphoreType.DMA((Z - 1,)),
            pltpu.SemaphoreType.DMA((Z - 1,)),
            pltpu.SemaphoreType.DMA((Z - 1,)),
            pltpu.SemaphoreType.DMA((Z - 1,)),
            pltpu.SemaphoreType.DMA((Z - 1,)),
        ],
        compiler_params=pltpu.CompilerParams(collective_id=0),
    )(x)


# device time: 22848 ns/iter; 1.1433x vs baseline; 1.1093x over previous
import jax
import jax.numpy as jnp
from jax import lax
from jax.experimental import pallas as pl
from jax.experimental.pallas import tpu as pltpu

Z = 4


def kernel(x):
    m, n = x.shape
    blk = n // Z
    qm = m // 4

    def body(x_ref, out_ref, xb_ref, zs_sems, zr_sems, ps_sems, pr_sems):
        my_x = lax.axis_index("x")
        my_y = lax.axis_index("y")
        my_z = lax.axis_index("z")
        q = 2 * my_y + my_x

        xb_ref[...] = x_ref[...].astype(jnp.bfloat16)

        peers = (
            (1 - my_x, my_y, my_z),
            (my_x, 1 - my_y, my_z),
            (1 - my_x, 1 - my_y, my_z),
        )

        barrier_sem = pltpu.get_barrier_semaphore()
        for d in range(1, Z):
            pl.semaphore_signal(
                barrier_sem, inc=1,
                device_id=(my_x, my_y, (my_z + d) % Z),
                device_id_type=pl.DeviceIdType.MESH,
            )
        for nbr in peers:
            pl.semaphore_signal(
                barrier_sem, inc=1,
                device_id=nbr, device_id_type=pl.DeviceIdType.MESH,
            )
        pl.semaphore_wait(barrier_sem, 6)

        z_rdmas = []
        for d in range(1, Z):
            tz = (my_z + d) % Z
            r = pltpu.make_async_remote_copy(
                src_ref=xb_ref.at[pl.ds(q * qm, qm), pl.ds(tz * blk, blk)],
                dst_ref=out_ref.at[pl.ds(my_z * m + q * qm, qm), :],
                send_sem=zs_sems.at[d - 1],
                recv_sem=zr_sems.at[d - 1],
                device_id=(my_x, my_y, tz),
                device_id_type=pl.DeviceIdType.MESH,
            )
            r.start()
            z_rdmas.append(r)

        out_ref[pl.ds(my_z * m, m), :] = xb_ref[:, pl.ds(my_z * blk, blk)]

        p_rdmas = []
        for d in range(1, Z):
            sz = (my_z - d) % Z
            z_rdmas[d - 1].wait_recv()
            rows = pl.ds(sz * m + q * qm, qm)
            for i, nbr in enumerate(peers):
                slot = (d - 1) * 3 + i
                r = pltpu.make_async_remote_copy(
                    src_ref=out_ref.at[rows, :],
                    dst_ref=out_ref.at[rows, :],
                    send_sem=ps_sems.at[slot],
                    recv_sem=pr_sems.at[slot],
                    device_id=nbr,
                    device_id_type=pl.DeviceIdType.MESH,
                )
                r.start()
                p_rdmas.append(r)

        for r in p_rdmas:
            r.wait_recv()
        for r in z_rdmas + p_rdmas:
            r.wait_send()

    out_shape = jax.ShapeDtypeStruct((Z * m, blk), jnp.bfloat16)
    return pl.pallas_call(
        body,
        out_shape=out_shape,
        in_specs=[pl.BlockSpec(memory_space=pltpu.VMEM)],
        out_specs=pl.BlockSpec(memory_space=pltpu.VMEM),
        scratch_shapes=[
            pltpu.VMEM((m, n), jnp.bfloat16),
            pltpu.SemaphoreType.DMA((Z - 1,)),
            pltpu.SemaphoreType.DMA((Z - 1,)),
            pltpu.SemaphoreType.DMA((3 * (Z - 1),)),
            pltpu.SemaphoreType.DMA((3 * (Z - 1),)),
        ],
        compiler_params=pltpu.CompilerParams(collective_id=0),
    )(x)
